# baseline (speedup 1.0000x reference)
import functools

import jax
import jax.numpy as jnp
from jax import lax
from jax.experimental import pallas as pl
from jax.experimental.pallas import tpu as pltpu
from jax.experimental.pallas import tpu_sc as plsc

N = 10000
E = 320000
D = 128
NC, NS = 2, 16
NW = NC * NS
CHUNK = 128
SG = 16
NSEG = 5
CPT = SG * NSEG
E_PAD = NW * CPT * CHUNK
RPT = 632
ACC_ROWS = NS * RPT
DUMP = N
NDUMP = ACC_ROWS - N
ZR = RPT // 4
DEG_W = 8
BLK = 400


def _make_sc_scatter():
    mesh = plsc.VectorSubcoreMesh(core_axis_name="c", subcore_axis_name="s")

    @functools.partial(
        pl.kernel,
        mesh=mesh,
        out_type=jax.ShapeDtypeStruct((NC, ACC_ROWS, D), jnp.float32),
        scratch_types=[
            pltpu.VMEM((SG, CHUNK), jnp.int32),
            pltpu.VMEM((SG, CHUNK), jnp.int32),
            pltpu.VMEM((CHUNK, D), jnp.float32),
            pltpu.VMEM((CHUNK, D), jnp.float32),
            pltpu.SemaphoreType.DMA,
            pltpu.SemaphoreType.DMA,
            pltpu.VMEM_SHARED((ACC_ROWS, D), jnp.float32),
        ],
    )
    def sc_scatter(g_hbm, src_hbm, dst_hbm, zrows_hbm, out_hbm,
                   src_v, dst_v, rows0, rows1, sem0, sem1, acc):
        c = lax.axis_index("c")
        s = lax.axis_index("s")
        wid = c * NS + s
        base = s * RPT
        for q in range(4):
            pltpu.sync_copy(zrows_hbm, acc.at[pl.ds(base + q * ZR, ZR)])
        plsc.subcore_barrier()

        def seg_body(seg, carry):
            pltpu.sync_copy(src_hbm.at[wid, seg], src_v)
            pltpu.sync_copy(dst_hbm.at[wid, seg], dst_v)
            pltpu.async_copy(g_hbm.at[src_v.at[0]], rows0, sem0)

            def body(jj, carry2):
                j = 2 * jj
                pltpu.async_copy(g_hbm.at[src_v.at[j + 1]], rows1, sem1)
                pltpu.make_async_copy(g_hbm.at[src_v.at[j]], rows0, sem0).wait()
                pltpu.sync_copy(rows0, acc.at[dst_v.at[j]], add=True)

                @pl.when(j + 2 < SG)
                def _():
                    pltpu.async_copy(g_hbm.at[src_v.at[j + 2]], rows0, sem0)

                pltpu.make_async_copy(g_hbm.at[src_v.at[j + 1]], rows1, sem1).wait()
                pltpu.sync_copy(rows1, acc.at[dst_v.at[j + 1]], add=True)
                return carry2

            lax.fori_loop(0, SG // 2, body, 0)
            return carry

        lax.fori_loop(0, NSEG, seg_body, 0)
        plsc.subcore_barrier()
        pltpu.sync_copy(acc.at[pl.ds(base, RPT)], out_hbm.at[c, pl.ds(base, RPT)])

    return sc_scatter


def _make_sc_degree():
    mesh = plsc.VectorSubcoreMesh(core_axis_name="c", subcore_axis_name="s")

    @functools.partial(
        pl.kernel,
        mesh=mesh,
        out_type=jax.ShapeDtypeStruct((NC, ACC_ROWS, D), jnp.float32),
        scratch_types=[
            pltpu.VMEM((CPT, CHUNK), jnp.int32),
            pltpu.VMEM((CHUNK, D), jnp.float32),
            pltpu.VMEM_SHARED((ACC_ROWS, D), jnp.float32),
        ],
    )
    def sc_degree(dst_hbm, ones_hbm, zrows_hbm, out_hbm, dst_v, ones_v, acc):
        c = lax.axis_index("c")
        s = lax.axis_index("s")
        wid = c * NS + s
        pltpu.sync_copy(dst_hbm.at[wid], dst_v)
        pltpu.sync_copy(ones_hbm, ones_v)
        base = s * RPT
        for q in range(4):
            pltpu.sync_copy(zrows_hbm, acc.at[pl.ds(base + q * ZR, ZR)])
        plsc.subcore_barrier()

        def body(j, carry):
            pltpu.sync_copy(ones_v, acc.at[dst_v.at[j]], add=True)
            return carry

        lax.fori_loop(0, CPT, body, 0)
        plsc.subcore_barrier()
        pltpu.sync_copy(acc.at[pl.ds(base, RPT)], out_hbm.at[c, pl.ds(base, RPT)])

    return sc_degree


_sc_scatter = _make_sc_scatter()
_sc_degree = _make_sc_degree()


def _row_spec():
    return pl.BlockSpec((BLK, D), lambda i: (i, 0))


def _deg_spec():
    return pl.BlockSpec((BLK, DEG_W), lambda i: (i, 0))


def _full_spec(shape):
    return pl.BlockSpec(shape, lambda i: (0,) * len(shape))


def _dinv(dp0_ref, dp1_ref):
    deg = dp0_ref[:, 0:1] + dp1_ref[:, 0:1] + 1.0
    return lax.rsqrt(deg)


def _tc_matmul_body(x, w, xw_out):
    xw_out[...] = jnp.dot(x[...], w[...], preferred_element_type=jnp.float32)


def _tc_scale_body(dp0, dp1, xw, g_out):
    g_out[...] = _dinv(dp0, dp1) * xw[...]


def _tc_mid_body(dp0, dp1, s0, s1, g, b, w, g_out):
    dinv = _dinv(dp0, dp1)
    h = dinv * (s0[...] + s1[...] + g[...]) + b[...]
    h = jax.nn.gelu(h)
    g_out[...] = dinv * jnp.dot(h, w[...], preferred_element_type=jnp.float32)


def _tc_last_body(dp0, dp1, s0, s1, g, b, out):
    out[...] = _dinv(dp0, dp1) * (s0[...] + s1[...] + g[...]) + b[...]


_GRID = (N // BLK,)
_OUT = jax.ShapeDtypeStruct((N, D), jnp.float32)

_tc_matmul = pl.pallas_call(
    _tc_matmul_body,
    grid=_GRID,
    in_specs=[_row_spec(), _full_spec((D, D))],
    out_specs=_row_spec(),
    out_shape=_OUT,
)

_tc_scale = pl.pallas_call(
    _tc_scale_body,
    grid=_GRID,
    in_specs=[_deg_spec(), _deg_spec(), _row_spec()],
    out_specs=_row_spec(),
    out_shape=_OUT,
)

_tc_mid = pl.pallas_call(
    _tc_mid_body,
    grid=_GRID,
    in_specs=[_deg_spec(), _deg_spec(), _row_spec(), _row_spec(), _row_spec(),
              _full_spec((1, D)), _full_spec((D, D))],
    out_specs=_row_spec(),
    out_shape=_OUT,
)

_tc_last = pl.pallas_call(
    _tc_last_body,
    grid=_GRID,
    in_specs=[_deg_spec(), _deg_spec(), _row_spec(), _row_spec(), _row_spec(),
              _full_spec((1, D))],
    out_specs=_row_spec(),
    out_shape=_OUT,
)


def kernel(x, edge_index, W1, b1, W2, b2, W3, b3):
    src = edge_index[0]
    dst = edge_index[1]
    pad = E_PAD - E
    pad_dst = DUMP + jnp.arange(pad, dtype=jnp.int32) % NDUMP
    src4 = jnp.concatenate([src, jnp.zeros((pad,), jnp.int32)]).reshape(NW, NSEG, SG, CHUNK)
    dst4 = jnp.concatenate([dst, pad_dst]).reshape(NW, NSEG, SG, CHUNK)
    dst3 = dst4.reshape(NW, CPT, CHUNK)
    zrows = jnp.zeros((ZR, D), jnp.float32)
    ones_rows = jnp.ones((CHUNK, D), jnp.float32)

    degp = _sc_degree(dst3, ones_rows, zrows)
    xw1 = _tc_matmul(x, W1)
    dp0 = degp[0, :N, :DEG_W]
    dp1 = degp[1, :N, :DEG_W]
    b1r, b2r, b3r = (b.reshape(1, D) for b in (b1, b2, b3))

    g1 = _tc_scale(dp0, dp1, xw1)
    S1 = _sc_scatter(g1, src4, dst4, zrows)
    g2 = _tc_mid(dp0, dp1, S1[0, :N], S1[1, :N], g1, b1r, W2)
    S2 = _sc_scatter(g2, src4, dst4, zrows)
    g3 = _tc_mid(dp0, dp1, S2[0, :N], S2[1, :N], g2, b2r, W3)
    S3 = _sc_scatter(g3, src4, dst4, zrows)
    return _tc_last(dp0, dp1, S3[0, :N], S3[1, :N], g3, b3r)


# speedup vs baseline: 2.5832x; 2.5832x over previous
import functools

import jax
import jax.numpy as jnp
from jax import lax
from jax.experimental import pallas as pl
from jax.experimental.pallas import tpu as pltpu
from jax.experimental.pallas import tpu_sc as plsc

N = 10000
E = 320000
D = 128
NC, NS = 2, 16
NW = NC * NS
CHUNK = 128
SG = 16
NSEG = 5
CPT = SG * NSEG
E_PAD = NW * CPT * CHUNK
RPT = 632
ACC_ROWS = NS * RPT
DUMP = N
NDUMP = ACC_ROWS - N
ZR = RPT // 4
DEG_W = 8
BLK = 400


def _make_sc_scatter():
    mesh = plsc.VectorSubcoreMesh(core_axis_name="c", subcore_axis_name="s")

    @functools.partial(
        pl.kernel,
        mesh=mesh,
        out_type=jax.ShapeDtypeStruct((NC, ACC_ROWS, D), jnp.float32),
        scratch_types=[
            pltpu.VMEM((SG, CHUNK), jnp.int32),
            pltpu.VMEM((SG, CHUNK), jnp.int32),
            pltpu.VMEM((CHUNK, D), jnp.float32),
            pltpu.VMEM((CHUNK, D), jnp.float32),
            pltpu.SemaphoreType.DMA,
            pltpu.SemaphoreType.DMA,
            pltpu.VMEM_SHARED((ACC_ROWS, D), jnp.float32),
        ],
    )
    def sc_scatter(g_hbm, src_hbm, dst_hbm, zrows_hbm, out_hbm,
                   src_v, dst_v, rows0, rows1, sem0, sem1, acc):
        c = lax.axis_index("c")
        s = lax.axis_index("s")
        wid = c * NS + s
        base = s * RPT
        pltpu.sync_copy(zrows_hbm, acc.at[pl.ds(base, RPT)])
        plsc.subcore_barrier()

        def seg_body(seg, carry):
            pltpu.sync_copy(src_hbm.at[wid, seg], src_v)
            pltpu.sync_copy(dst_hbm.at[wid, seg], dst_v)
            pltpu.async_copy(g_hbm.at[src_v.at[0]], rows0, sem0)

            def body(jj, carry2):
                j = 2 * jj
                pltpu.async_copy(g_hbm.at[src_v.at[j + 1]], rows1, sem1)
                pltpu.make_async_copy(g_hbm.at[src_v.at[j]], rows0, sem0).wait()
                pltpu.sync_copy(rows0, acc.at[dst_v.at[j]], add=True)

                @pl.when(j + 2 < SG)
                def _():
                    pltpu.async_copy(g_hbm.at[src_v.at[j + 2]], rows0, sem0)

                pltpu.make_async_copy(g_hbm.at[src_v.at[j + 1]], rows1, sem1).wait()
                pltpu.sync_copy(rows1, acc.at[dst_v.at[j + 1]], add=True)
                return carry2

            lax.fori_loop(0, SG // 2, body, 0)
            return carry

        lax.fori_loop(0, NSEG, seg_body, 0)
        plsc.subcore_barrier()
        pltpu.sync_copy(acc.at[pl.ds(base, RPT)], out_hbm.at[c, pl.ds(base, RPT)])

    return sc_scatter


def _make_sc_degree():
    mesh = plsc.VectorSubcoreMesh(core_axis_name="c", subcore_axis_name="s")

    @functools.partial(
        pl.kernel,
        mesh=mesh,
        out_type=jax.ShapeDtypeStruct((NC, ACC_ROWS, D), jnp.float32),
        scratch_types=[
            pltpu.VMEM((CPT, CHUNK), jnp.int32),
            pltpu.VMEM((CHUNK, D), jnp.float32),
            pltpu.VMEM_SHARED((ACC_ROWS, D), jnp.float32),
        ],
    )
    def sc_degree(dst_hbm, ones_hbm, zrows_hbm, out_hbm, dst_v, ones_v, acc):
        c = lax.axis_index("c")
        s = lax.axis_index("s")
        wid = c * NS + s
        pltpu.sync_copy(dst_hbm.at[wid], dst_v)
        pltpu.sync_copy(ones_hbm, ones_v)
        base = s * RPT
        pltpu.sync_copy(zrows_hbm, acc.at[pl.ds(base, RPT)])
        plsc.subcore_barrier()

        def body(j, carry):
            pltpu.sync_copy(ones_v, acc.at[dst_v.at[j]], add=True)
            return carry

        lax.fori_loop(0, CPT, body, 0)
        plsc.subcore_barrier()
        pltpu.sync_copy(acc.at[pl.ds(base, RPT)], out_hbm.at[c, pl.ds(base, RPT)])

    return sc_degree


_sc_scatter = _make_sc_scatter()
_sc_degree = _make_sc_degree()


def _row_spec():
    return pl.BlockSpec((BLK, D), lambda i: (i, 0))


def _deg_spec():
    return pl.BlockSpec((BLK, DEG_W), lambda i: (i, 0))


def _full_spec(shape):
    return pl.BlockSpec(shape, lambda i: (0,) * len(shape))


def _dinv(dp0_ref, dp1_ref):
    deg = dp0_ref[:, 0:1] + dp1_ref[:, 0:1] + 1.0
    return lax.rsqrt(deg)


def _tc_matmul_body(x, w, xw_out):
    xw_out[...] = jnp.dot(x[...], w[...], preferred_element_type=jnp.float32)


def _tc_scale_body(dp0, dp1, xw, g_out):
    g_out[...] = _dinv(dp0, dp1) * xw[...]


def _tc_mid_body(dp0, dp1, s0, s1, g, b, w, g_out):
    dinv = _dinv(dp0, dp1)
    h = dinv * (s0[...] + s1[...] + g[...]) + b[...]
    h = jax.nn.gelu(h)
    g_out[...] = dinv * jnp.dot(h, w[...], preferred_element_type=jnp.float32)


def _tc_last_body(dp0, dp1, s0, s1, g, b, out):
    out[...] = _dinv(dp0, dp1) * (s0[...] + s1[...] + g[...]) + b[...]


_GRID = (N // BLK,)
_OUT = jax.ShapeDtypeStruct((N, D), jnp.float32)

_tc_matmul = pl.pallas_call(
    _tc_matmul_body,
    grid=_GRID,
    in_specs=[_row_spec(), _full_spec((D, D))],
    out_specs=_row_spec(),
    out_shape=_OUT,
)

_tc_scale = pl.pallas_call(
    _tc_scale_body,
    grid=_GRID,
    in_specs=[_deg_spec(), _deg_spec(), _row_spec()],
    out_specs=_row_spec(),
    out_shape=_OUT,
)

_tc_mid = pl.pallas_call(
    _tc_mid_body,
    grid=_GRID,
    in_specs=[_deg_spec(), _deg_spec(), _row_spec(), _row_spec(), _row_spec(),
              _full_spec((1, D)), _full_spec((D, D))],
    out_specs=_row_spec(),
    out_shape=_OUT,
)

_tc_last = pl.pallas_call(
    _tc_last_body,
    grid=_GRID,
    in_specs=[_deg_spec(), _deg_spec(), _row_spec(), _row_spec(), _row_spec(),
              _full_spec((1, D))],
    out_specs=_row_spec(),
    out_shape=_OUT,
)


def kernel(x, edge_index, W1, b1, W2, b2, W3, b3):
    src = edge_index[0]
    dst = edge_index[1]
    pad = E_PAD - E
    pad_ar = jnp.arange(pad, dtype=jnp.int32)
    pad_dst = DUMP + pad_ar % NDUMP
    pad_src = pad_ar % N
    src4 = jnp.concatenate([src, pad_src]).reshape(NW, NSEG, SG, CHUNK)
    dst4 = jnp.concatenate([dst, pad_dst]).reshape(NW, NSEG, SG, CHUNK)
    dst3 = dst4.reshape(NW, CPT, CHUNK)
    zrows = jnp.zeros((RPT, D), jnp.float32)
    ones_rows = jnp.ones((CHUNK, D), jnp.float32)

    degp = _sc_degree(dst3, ones_rows, zrows)
    xw1 = _tc_matmul(x, W1)
    dp0 = degp[0, :N, :DEG_W]
    dp1 = degp[1, :N, :DEG_W]
    b1r, b2r, b3r = (b.reshape(1, D) for b in (b1, b2, b3))

    g1 = _tc_scale(dp0, dp1, xw1)
    S1 = _sc_scatter(g1, src4, dst4, zrows)
    g2 = _tc_mid(dp0, dp1, S1[0, :N], S1[1, :N], g1, b1r, W2)
    S2 = _sc_scatter(g2, src4, dst4, zrows)
    g3 = _tc_mid(dp0, dp1, S2[0, :N], S2[1, :N], g2, b2r, W3)
    S3 = _sc_scatter(g3, src4, dst4, zrows)
    return _tc_last(dp0, dp1, S3[0, :N], S3[1, :N], g3, b3r)


# speedup vs baseline: 2.7546x; 1.0663x over previous
import functools

import jax
import jax.numpy as jnp
from jax import lax
from jax.experimental import pallas as pl
from jax.experimental.pallas import tpu as pltpu
from jax.experimental.pallas import tpu_sc as plsc

N = 10000
E = 320000
D = 128
NC, NS = 2, 16
NW = NC * NS
CHUNK = 64
NBUF = 4
SG = 32
NSEG = 5
CPT = SG * NSEG
E_PAD = NW * CPT * CHUNK
CH_DEG = 128
CPT_DEG = 80
RPT = 632
ACC_ROWS = NS * RPT
DUMP = N
NDUMP = ACC_ROWS - N
ZR = RPT // 4
DEG_W = 8
BLK = 400


def _make_sc_scatter():
    mesh = plsc.VectorSubcoreMesh(core_axis_name="c", subcore_axis_name="s")

    @functools.partial(
        pl.kernel,
        mesh=mesh,
        out_type=jax.ShapeDtypeStruct((NC, ACC_ROWS, D), jnp.float32),
        scratch_types=[
            pltpu.VMEM((SG, CHUNK), jnp.int32),
            pltpu.VMEM((SG, CHUNK), jnp.int32),
        ] + [pltpu.VMEM((CHUNK, D), jnp.float32)] * NBUF
          + [pltpu.SemaphoreType.DMA] * NBUF + [
            pltpu.VMEM_SHARED((ACC_ROWS, D), jnp.float32),
        ],
    )
    def sc_scatter(g_hbm, src_hbm, dst_hbm, zrows_hbm, out_hbm,
                   src_v, dst_v, *rest):
        rows = rest[:NBUF]
        sems = rest[NBUF:2 * NBUF]
        acc = rest[2 * NBUF]
        c = lax.axis_index("c")
        s = lax.axis_index("s")
        wid = c * NS + s
        base = s * RPT
        pltpu.sync_copy(zrows_hbm, acc.at[pl.ds(base, RPT)])
        plsc.subcore_barrier()

        def seg_body(seg, carry):
            pltpu.sync_copy(src_hbm.at[wid, seg], src_v)
            pltpu.sync_copy(dst_hbm.at[wid, seg], dst_v)
            for b in range(NBUF - 1):
                pltpu.async_copy(g_hbm.at[src_v.at[b]], rows[b], sems[b])

            def body(jj, carry2):
                for b in range(NBUF):
                    j = jj * NBUF + b
                    nxt = j + NBUF - 1
                    bn = (b + NBUF - 1) % NBUF

                    @pl.when(nxt < SG)
                    def _():
                        pltpu.async_copy(g_hbm.at[src_v.at[nxt]], rows[bn], sems[bn])

                    pltpu.make_async_copy(g_hbm.at[src_v.at[j]], rows[b], sems[b]).wait()
                    pltpu.sync_copy(rows[b], acc.at[dst_v.at[j]], add=True)
                return carry2

            lax.fori_loop(0, SG // NBUF, body, 0)
            return carry

        lax.fori_loop(0, NSEG, seg_body, 0)
        plsc.subcore_barrier()
        pltpu.sync_copy(acc.at[pl.ds(base, RPT)], out_hbm.at[c, pl.ds(base, RPT)])

    return sc_scatter


def _make_sc_degree():
    mesh = plsc.VectorSubcoreMesh(core_axis_name="c", subcore_axis_name="s")

    @functools.partial(
        pl.kernel,
        mesh=mesh,
        out_type=jax.ShapeDtypeStruct((NC, ACC_ROWS, D), jnp.float32),
        scratch_types=[
            pltpu.VMEM((CPT_DEG, CH_DEG), jnp.int32),
            pltpu.VMEM((CH_DEG, D), jnp.float32),
            pltpu.VMEM_SHARED((ACC_ROWS, D), jnp.float32),
        ],
    )
    def sc_degree(dst_hbm, ones_hbm, zrows_hbm, out_hbm, dst_v, ones_v, acc):
        c = lax.axis_index("c")
        s = lax.axis_index("s")
        wid = c * NS + s
        pltpu.sync_copy(dst_hbm.at[wid], dst_v)
        pltpu.sync_copy(ones_hbm, ones_v)
        base = s * RPT
        pltpu.sync_copy(zrows_hbm, acc.at[pl.ds(base, RPT)])
        plsc.subcore_barrier()

        def body(j, carry):
            pltpu.sync_copy(ones_v, acc.at[dst_v.at[j]], add=True)
            return carry

        lax.fori_loop(0, CPT_DEG, body, 0)
        plsc.subcore_barrier()
        pltpu.sync_copy(acc.at[pl.ds(base, RPT)], out_hbm.at[c, pl.ds(base, RPT)])

    return sc_degree


_sc_scatter = _make_sc_scatter()
_sc_degree = _make_sc_degree()


def _row_spec():
    return pl.BlockSpec((BLK, D), lambda i: (i, 0))


def _deg_spec():
    return pl.BlockSpec((BLK, DEG_W), lambda i: (i, 0))


def _full_spec(shape):
    return pl.BlockSpec(shape, lambda i: (0,) * len(shape))


def _dinv(dp0_ref, dp1_ref):
    deg = dp0_ref[:, 0:1] + dp1_ref[:, 0:1] + 1.0
    return lax.rsqrt(deg)


def _tc_matmul_body(x, w, xw_out):
    xw_out[...] = jnp.dot(x[...], w[...], preferred_element_type=jnp.float32)


def _tc_scale_body(dp0, dp1, xw, g_out):
    g_out[...] = _dinv(dp0, dp1) * xw[...]


def _tc_mid_body(dp0, dp1, s0, s1, g, b, w, g_out):
    dinv = _dinv(dp0, dp1)
    h = dinv * (s0[...] + s1[...] + g[...]) + b[...]
    h = jax.nn.gelu(h)
    g_out[...] = dinv * jnp.dot(h, w[...], preferred_element_type=jnp.float32)


def _tc_last_body(dp0, dp1, s0, s1, g, b, out):
    out[...] = _dinv(dp0, dp1) * (s0[...] + s1[...] + g[...]) + b[...]


_GRID = (N // BLK,)
_OUT = jax.ShapeDtypeStruct((N, D), jnp.float32)

_tc_matmul = pl.pallas_call(
    _tc_matmul_body,
    grid=_GRID,
    in_specs=[_row_spec(), _full_spec((D, D))],
    out_specs=_row_spec(),
    out_shape=_OUT,
)

_tc_scale = pl.pallas_call(
    _tc_scale_body,
    grid=_GRID,
    in_specs=[_deg_spec(), _deg_spec(), _row_spec()],
    out_specs=_row_spec(),
    out_shape=_OUT,
)

_tc_mid = pl.pallas_call(
    _tc_mid_body,
    grid=_GRID,
    in_specs=[_deg_spec(), _deg_spec(), _row_spec(), _row_spec(), _row_spec(),
              _full_spec((1, D)), _full_spec((D, D))],
    out_specs=_row_spec(),
    out_shape=_OUT,
)

_tc_last = pl.pallas_call(
    _tc_last_body,
    grid=_GRID,
    in_specs=[_deg_spec(), _deg_spec(), _row_spec(), _row_spec(), _row_spec(),
              _full_spec((1, D))],
    out_specs=_row_spec(),
    out_shape=_OUT,
)


def kernel(x, edge_index, W1, b1, W2, b2, W3, b3):
    src = edge_index[0]
    dst = edge_index[1]
    pad = E_PAD - E
    pad_ar = jnp.arange(pad, dtype=jnp.int32)
    pad_dst = DUMP + pad_ar % NDUMP
    pad_src = pad_ar % N
    src4 = jnp.concatenate([src, pad_src]).reshape(NW, NSEG, SG, CHUNK)
    dst_p = jnp.concatenate([dst, pad_dst])
    dst4 = dst_p.reshape(NW, NSEG, SG, CHUNK)
    dst3 = dst_p.reshape(NW, CPT_DEG, CH_DEG)
    zrows = jnp.zeros((RPT, D), jnp.float32)
    ones_rows = jnp.ones((CH_DEG, D), jnp.float32)

    degp = _sc_degree(dst3, ones_rows, zrows)
    xw1 = _tc_matmul(x, W1)
    dp0 = degp[0, :N, :DEG_W]
    dp1 = degp[1, :N, :DEG_W]
    b1r, b2r, b3r = (b.reshape(1, D) for b in (b1, b2, b3))

    g1 = _tc_scale(dp0, dp1, xw1)
    S1 = _sc_scatter(g1, src4, dst4, zrows)
    g2 = _tc_mid(dp0, dp1, S1[0, :N], S1[1, :N], g1, b1r, W2)
    S2 = _sc_scatter(g2, src4, dst4, zrows)
    g3 = _tc_mid(dp0, dp1, S2[0, :N], S2[1, :N], g2, b2r, W3)
    S3 = _sc_scatter(g3, src4, dst4, zrows)
    return _tc_last(dp0, dp1, S3[0, :N], S3[1, :N], g3, b3r)
